# baseline (speedup 1.0000x reference)
import functools

import jax
import jax.numpy as jnp
from jax import lax
from jax.experimental import pallas as pl
from jax.experimental.pallas import tpu as pltpu
from jax.experimental.pallas import tpu_sc as plsc

_BLANK = 0
_W = 8
_NEG_INF = -1.0e30
_PAD = -2.0e30
_MASKED = -3.0e30
_BIGI = 1 << 20
_T = 512
_B = 32
_C = 64


def _fwd_body(data_ref, dl_ref, bp_ref, best_ref, lpb_s, lpnb_s, lens_s, last_s):
    t = pl.program_id(0)

    @pl.when(t == 0)
    def _init():
        lane = jax.lax.broadcasted_iota(jnp.int32, (_B, _W), 1)
        lpb_s[...] = jnp.where(lane == 0, 0.0, _NEG_INF).astype(jnp.float32)
        lpnb_s[...] = jnp.full((_B, _W), _NEG_INF, jnp.float32)
        lens_s[...] = jnp.zeros((_B, _W), jnp.int32)
        last_s[...] = jnp.zeros((_B, _W), jnp.int32)

    lp = data_ref[0]
    lpb = lpb_s[...]
    lpnb = lpnb_s[...]
    lens = lens_s[...]
    last = last_s[...]

    iotac = jax.lax.broadcasted_iota(jnp.int32, (_B, _C), 1)
    tot = jnp.logaddexp(lpb, lpnb)
    lp0 = lp[:, 0:1]
    stay_lpb = tot + lp0

    ext_pieces = []
    lp_last_cols = []
    for w in range(_W):
        last_w = last[:, w : w + 1]
        match_w = iotac == last_w
        hp_w = lens[:, w : w + 1] > 0
        sel = jnp.where(match_w & hp_w, lpb[:, w : w + 1], tot[:, w : w + 1])
        piece = jnp.where(iotac == _BLANK, _NEG_INF, sel + lp)
        ext_pieces.append(piece)
        lp_last_cols.append(
            jnp.sum(jnp.where(match_w, lp, 0.0), axis=1, keepdims=True))
    ext = jnp.concatenate(ext_pieces, axis=1)
    lp_last = jnp.concatenate(lp_last_cols, axis=1)

    stay_lpnb = jnp.where(lens > 0, lpnb + lp_last, _NEG_INF)
    stay_score = jnp.logaddexp(stay_lpb, stay_lpnb)

    pad = jnp.full((_B, 120), _PAD, jnp.float32)
    cand = jnp.concatenate([stay_score, ext, pad], axis=1)
    fidx = jax.lax.broadcasted_iota(jnp.int32, (_B, 640), 1)

    def lex_combine(va, fa, vb, fb):
        take_b = (vb > va) | ((vb == va) & (fb < fa))
        return jnp.where(take_b, vb, va), jnp.where(take_b, fb, fa)

    sel_v, sel_f = [], []
    v = cand
    for _ in range(_W):
        cv, cf = v[:, 0:128], fidx[:, 0:128]
        for tile in range(1, 5):
            cv, cf = lex_combine(cv, cf, v[:, tile * 128:(tile + 1) * 128],
                                 fidx[:, tile * 128:(tile + 1) * 128])
        for sh in (1, 2, 4, 8, 16, 32, 64):
            rv = pltpu.roll(cv, sh, 1)
            rf = pltpu.roll(cf, sh, 1)
            cv, cf = lex_combine(cv, cf, rv, rf)
        m = cv[:, 0:1]
        fk = cf[:, 0:1]
        sel_v.append(m)
        sel_f.append(fk)
        v = jnp.where(fidx == fk, _MASKED, v)
    top_v = jnp.concatenate(sel_v, axis=1)
    top_f = jnp.concatenate(sel_f, axis=1)

    is_stay = top_f < _W
    src = jnp.where(is_stay, top_f, (top_f - _W) >> 6)
    cls = jnp.where(is_stay, 0, (top_f - _W) & 63)

    def sel8(s, arr):
        acc = arr[:, 0:1]
        for w in range(1, _W):
            acc = jnp.where(s == w, arr[:, w : w + 1], acc)
        return acc

    g_staylpb = sel8(src, stay_lpb)
    g_staylpnb = sel8(src, stay_lpnb)
    g_lens = sel8(src, lens)
    g_last = sel8(src, last)

    new_lpb = jnp.where(is_stay, g_staylpb, _NEG_INF)
    new_lpnb = jnp.where(is_stay, g_staylpnb, top_v)
    new_lens = jnp.where(is_stay, g_lens, jnp.minimum(g_lens + 1, _T))
    new_last = jnp.where(is_stay, g_last, cls)

    active = t < dl_ref[...]
    lpb_s[...] = jnp.where(active, new_lpb, lpb)
    lpnb_s[...] = jnp.where(active, new_lpnb, lpnb)
    lens_s[...] = jnp.where(active, new_lens, lens)
    last_s[...] = jnp.where(active, new_last, last)

    bp_ref[...] = ((src << 8) | (cls << 1) | is_stay.astype(jnp.int32)).reshape(1, _B, _W)

    @pl.when(t == _T - 1)
    def _final():
        fscore = jnp.logaddexp(lpb_s[...], lpnb_s[...])
        m = jnp.max(fscore, axis=1, keepdims=True)
        widx = jax.lax.broadcasted_iota(jnp.int32, (_B, _W), 1)
        bw = jnp.min(jnp.where(fscore == m, widx, _BIGI), axis=1, keepdims=True)
        blen = sel8(bw, lens_s[...])
        zpad = jnp.zeros((_B, 13), jnp.int32)
        best_ref[...] = jnp.concatenate(
            [bw, blen, dl_ref[...][:, 0:1], zpad], axis=1)


def _forward(data, data_length):
    dl2 = jnp.broadcast_to(data_length[:, None], (_B, _W))
    bp, best = pl.pallas_call(
        _fwd_body,
        grid=(_T,),
        in_specs=[
            pl.BlockSpec((1, _B, _C), lambda t: (t, 0, 0)),
            pl.BlockSpec((_B, _W), lambda t: (0, 0)),
        ],
        out_specs=[
            pl.BlockSpec((1, _B, _W), lambda t: (t, 0, 0)),
            pl.BlockSpec((_B, 16), lambda t: (0, 0)),
        ],
        out_shape=[
            jax.ShapeDtypeStruct((_T, _B, _W), jnp.int32),
            jax.ShapeDtypeStruct((_B, 16), jnp.int32),
        ],
        scratch_shapes=[
            pltpu_vmem((_B, _W), jnp.float32),
            pltpu_vmem((_B, _W), jnp.float32),
            pltpu_vmem((_B, _W), jnp.int32),
            pltpu_vmem((_B, _W), jnp.int32),
        ],
    )(data, dl2)
    return bp, best


def pltpu_vmem(shape, dtype):
    from jax.experimental.pallas import tpu as pltpu
    return pltpu.VMEM(shape, dtype)


def _sc_backtrace(bp2, best):
    mesh = plsc.VectorSubcoreMesh(core_axis_name="c", subcore_axis_name="s")

    @functools.partial(
        pl.kernel,
        mesh=mesh,
        out_type=jax.ShapeDtypeStruct((_B, _T), jnp.int32),
        compiler_params=pltpu.CompilerParams(needs_layout_passes=False),
        scratch_types=[
            pltpu.VMEM((_T * _W,), jnp.int32),
            pltpu.VMEM((528,), jnp.int32),
            pltpu.VMEM((16,), jnp.int32),
        ],
    )
    def bt(bp_hbm, info_hbm, out_hbm, bpv, outv, infov):
        b = lax.axis_index("s") * 2 + lax.axis_index("c")
        pltpu.sync_copy(bp_hbm.at[b], bpv)
        pltpu.sync_copy(info_hbm.at[b], infov)
        z = jnp.zeros((16,), jnp.int32)
        for i in range(33):
            outv[pl.ds(i * 16, 16)] = z
        iv = infov[...]
        bw = iv[0]
        blen = iv[1]
        ln = iv[2]
        lane0 = lax.iota(jnp.int32, 16) == 0

        def body(i, carry):
            w, pos = carry
            t = _T - 1 - i
            idxv = jnp.full((16,), t * _W + w, jnp.int32)
            e = plsc.load_gather(bpv, [idxv])[0]
            act = t < ln
            stay = (e & 1) == 1
            srcb = e >> 8
            clsb = (e >> 1) & 63
            do_write = act & (~stay)
            oidx = jnp.where(do_write, pos, _T)
            plsc.store_scatter(outv, [jnp.full((16,), oidx, jnp.int32)],
                               jnp.full((16,), clsb, jnp.int32), mask=lane0)
            pos = pos - do_write.astype(jnp.int32)
            w = jnp.where(act, srcb, w)
            return (w, pos)

        lax.fori_loop(0, _T, body, (bw, blen - 1))
        pltpu.sync_copy(outv.at[pl.ds(0, _T)], out_hbm.at[b])

    return bt(bp2, best)


def _backtrace_plain(bp, best):
    bw = best[:, 0]
    blen = best[:, 1]
    length = best[:, 2]
    bseq = jnp.transpose(bp, (1, 0, 2))

    def body(i, carry):
        w, pos, out = carry
        t = _T - 1 - i
        e = bseq[jnp.arange(_B), t, w]
        act = t < length
        stay = (e & 1) == 1
        srcb = e >> 8
        clsb = (e >> 1) & 63
        do_write = act & (~stay)
        out = jnp.where(
            do_write[:, None] & (jnp.arange(_T)[None, :] == pos[:, None]),
            clsb[:, None], out)
        pos = pos - do_write.astype(jnp.int32)
        w = jnp.where(act, srcb, w)
        return (w, pos, out)

    out0 = jnp.zeros((_B, _T), jnp.int32)
    _, _, out = jax.lax.fori_loop(0, _T, body, (bw, blen - 1, out0))
    return out


def kernel(data, data_length):
    bp, best = _forward(data, data_length)
    bp2 = jnp.transpose(bp, (1, 0, 2)).reshape(_B, _T * _W)
    return _sc_backtrace(bp2, best)


# speedup vs baseline: 1.8641x; 1.8641x over previous
import functools

import jax
import jax.numpy as jnp
from jax import lax
from jax.experimental import pallas as pl
from jax.experimental.pallas import tpu as pltpu
from jax.experimental.pallas import tpu_sc as plsc

_BLANK = 0
_W = 8
_NEG_INF = -1.0e30
_PAD = -2.0e30
_MASKED = -3.0e30
_BIGI = 1 << 20
_T = 512
_B = 32
_C = 64


_K = 13
_TB = 8


def _pre_body(data_ref, kv_ref, ki_ref):
    x = data_ref[...].reshape(_TB * _B, _C)
    iotac = jax.lax.broadcasted_iota(jnp.int32, (_TB * _B, _C), 1)
    x = jnp.where(iotac == _BLANK, _MASKED, x)
    kv_cols, ki_cols = [], []
    for _ in range(_K):
        m = jnp.max(x, axis=1, keepdims=True)
        ci = jnp.min(jnp.where(x == m, iotac, _BIGI), axis=1, keepdims=True)
        kv_cols.append(m)
        ki_cols.append(ci)
        x = jnp.where(iotac == ci, _MASKED, x)
    for _ in range(16 - _K):
        kv_cols.append(jnp.full((_TB * _B, 1), _PAD, jnp.float32))
        ki_cols.append(jnp.zeros((_TB * _B, 1), jnp.int32))
    kv_ref[...] = jnp.concatenate(kv_cols, axis=1).reshape(_TB, _B, 16)
    ki_ref[...] = jnp.concatenate(ki_cols, axis=1).reshape(_TB, _B, 16)


def _prepass(data):
    return pl.pallas_call(
        _pre_body,
        grid=(_T // _TB,),
        in_specs=[pl.BlockSpec((_TB, _B, _C), lambda t: (t, 0, 0))],
        out_specs=[
            pl.BlockSpec((_TB, _B, 16), lambda t: (t, 0, 0)),
            pl.BlockSpec((_TB, _B, 16), lambda t: (t, 0, 0)),
        ],
        out_shape=[
            jax.ShapeDtypeStruct((_T, _B, 16), jnp.float32),
            jax.ShapeDtypeStruct((_T, _B, 16), jnp.int32),
        ],
    )(data)


def _fwd_body(data_ref, dl_ref, kv_ref, ki_ref, bp_ref, best_ref,
              lpb_s, lpnb_s, lens_s, last_s):
    t = pl.program_id(0)

    @pl.when(t == 0)
    def _init():
        lane = jax.lax.broadcasted_iota(jnp.int32, (_B, _W), 1)
        lpb_s[...] = jnp.where(lane == 0, 0.0, _NEG_INF).astype(jnp.float32)
        lpnb_s[...] = jnp.full((_B, _W), _NEG_INF, jnp.float32)
        lens_s[...] = jnp.zeros((_B, _W), jnp.int32)
        last_s[...] = jnp.zeros((_B, _W), jnp.int32)

    lp = data_ref[0]
    lpb = lpb_s[...]
    lpnb = lpnb_s[...]
    lens = lens_s[...]
    last = last_s[...]

    kv = kv_ref[0]
    ki = ki_ref[0]

    iotac = jax.lax.broadcasted_iota(jnp.int32, (_B, _C), 1)
    hp = lens > 0
    tot = jnp.logaddexp(lpb, lpnb)
    lp0 = lp[:, 0:1]
    stay_lpb = tot + lp0

    lp_last_cols = []
    for w in range(_W):
        match_w = iotac == last[:, w : w + 1]
        lp_last_cols.append(
            jnp.sum(jnp.where(match_w, lp, 0.0), axis=1, keepdims=True))
    lp_last = jnp.concatenate(lp_last_cols, axis=1)

    stay_lpnb = jnp.where(hp, lpnb + lp_last, _NEG_INF)
    stay_score = jnp.logaddexp(stay_lpb, stay_lpnb)

    padv8 = jnp.full((_B, _W), _PAD, jnp.float32)
    bigf8 = jnp.full((_B, _W), _BIGI, jnp.int32)
    pieces_v = [jnp.concatenate([stay_score, padv8], axis=1)]
    w_iota8 = jax.lax.broadcasted_iota(jnp.int32, (_B, _W), 1)
    pieces_f = [jnp.concatenate([w_iota8, bigf8 + w_iota8], axis=1)]
    for w in range(_W):
        dup = (ki == last[:, w : w + 1]) & hp[:, w : w + 1]
        vw = jnp.where(dup, _PAD, tot[:, w : w + 1] + kv)
        pieces_v.append(vw)
        pieces_f.append(8 + w * _C + ki)
    corr_v = jnp.where(hp, lpb + lp_last, _PAD)
    corr_f = jnp.where(hp, 8 + w_iota8 * _C + last, 2 * _BIGI + w_iota8)
    pieces_v.append(jnp.concatenate([corr_v, padv8], axis=1))
    pieces_f.append(jnp.concatenate([corr_f, 3 * _BIGI + w_iota8], axis=1))
    cand = jnp.concatenate(pieces_v, axis=1)
    fidx = jnp.concatenate(pieces_f, axis=1)

    sel_v, sel_f = [], []
    v = cand
    for _ in range(_W):
        m = jnp.max(v, axis=1, keepdims=True)
        fk = jnp.min(jnp.where(v == m, fidx, _BIGI), axis=1, keepdims=True)
        sel_v.append(m)
        sel_f.append(fk)
        v = jnp.where(fidx == fk, _MASKED, v)
    top_v = jnp.concatenate(sel_v, axis=1)
    top_f = jnp.concatenate(sel_f, axis=1)

    is_stay = top_f < _W
    src = jnp.where(is_stay, top_f, (top_f - _W) >> 6)
    cls = jnp.where(is_stay, 0, (top_f - _W) & 63)

    def sel8(s, arr):
        acc = arr[:, 0:1]
        for w in range(1, _W):
            acc = jnp.where(s == w, arr[:, w : w + 1], acc)
        return acc

    g_staylpb = sel8(src, stay_lpb)
    g_staylpnb = sel8(src, stay_lpnb)
    g_lens = sel8(src, lens)
    g_last = sel8(src, last)

    new_lpb = jnp.where(is_stay, g_staylpb, _NEG_INF)
    new_lpnb = jnp.where(is_stay, g_staylpnb, top_v)
    new_lens = jnp.where(is_stay, g_lens, jnp.minimum(g_lens + 1, _T))
    new_last = jnp.where(is_stay, g_last, cls)

    active = t < dl_ref[...]
    lpb_s[...] = jnp.where(active, new_lpb, lpb)
    lpnb_s[...] = jnp.where(active, new_lpnb, lpnb)
    lens_s[...] = jnp.where(active, new_lens, lens)
    last_s[...] = jnp.where(active, new_last, last)

    bp_ref[...] = ((src << 8) | (cls << 1) | is_stay.astype(jnp.int32)).reshape(1, _B, _W)

    @pl.when(t == _T - 1)
    def _final():
        fscore = jnp.logaddexp(lpb_s[...], lpnb_s[...])
        m = jnp.max(fscore, axis=1, keepdims=True)
        widx = jax.lax.broadcasted_iota(jnp.int32, (_B, _W), 1)
        bw = jnp.min(jnp.where(fscore == m, widx, _BIGI), axis=1, keepdims=True)
        blen = sel8(bw, lens_s[...])
        zpad = jnp.zeros((_B, 13), jnp.int32)
        best_ref[...] = jnp.concatenate(
            [bw, blen, dl_ref[...][:, 0:1], zpad], axis=1)


def _forward(data, data_length):
    dl2 = jnp.broadcast_to(data_length[:, None], (_B, _W))
    kv, ki = _prepass(data)
    bp, best = pl.pallas_call(
        _fwd_body,
        grid=(_T,),
        in_specs=[
            pl.BlockSpec((1, _B, _C), lambda t: (t, 0, 0)),
            pl.BlockSpec((_B, _W), lambda t: (0, 0)),
            pl.BlockSpec((1, _B, 16), lambda t: (t, 0, 0)),
            pl.BlockSpec((1, _B, 16), lambda t: (t, 0, 0)),
        ],
        out_specs=[
            pl.BlockSpec((1, _B, _W), lambda t: (t, 0, 0)),
            pl.BlockSpec((_B, 16), lambda t: (0, 0)),
        ],
        out_shape=[
            jax.ShapeDtypeStruct((_T, _B, _W), jnp.int32),
            jax.ShapeDtypeStruct((_B, 16), jnp.int32),
        ],
        scratch_shapes=[
            pltpu_vmem((_B, _W), jnp.float32),
            pltpu_vmem((_B, _W), jnp.float32),
            pltpu_vmem((_B, _W), jnp.int32),
            pltpu_vmem((_B, _W), jnp.int32),
        ],
    )(data, dl2, kv, ki)
    return bp, best


def pltpu_vmem(shape, dtype):
    from jax.experimental.pallas import tpu as pltpu
    return pltpu.VMEM(shape, dtype)


def _sc_backtrace(bp2, best):
    mesh = plsc.VectorSubcoreMesh(core_axis_name="c", subcore_axis_name="s")

    @functools.partial(
        pl.kernel,
        mesh=mesh,
        out_type=jax.ShapeDtypeStruct((_B, _T), jnp.int32),
        compiler_params=pltpu.CompilerParams(needs_layout_passes=False),
        scratch_types=[
            pltpu.VMEM((_T * _W,), jnp.int32),
            pltpu.VMEM((528,), jnp.int32),
            pltpu.VMEM((16,), jnp.int32),
        ],
    )
    def bt(bp_hbm, info_hbm, out_hbm, bpv, outv, infov):
        b = lax.axis_index("s") * 2 + lax.axis_index("c")
        pltpu.sync_copy(bp_hbm.at[b], bpv)
        pltpu.sync_copy(info_hbm.at[b], infov)
        z = jnp.zeros((16,), jnp.int32)
        for i in range(33):
            outv[pl.ds(i * 16, 16)] = z
        iv = infov[...]
        bw = iv[0]
        blen = iv[1]
        ln = iv[2]
        lane0 = lax.iota(jnp.int32, 16) == 0

        def body(i, carry):
            w, pos = carry
            t = _T - 1 - i
            idxv = jnp.full((16,), t * _W + w, jnp.int32)
            e = plsc.load_gather(bpv, [idxv])[0]
            act = t < ln
            stay = (e & 1) == 1
            srcb = e >> 8
            clsb = (e >> 1) & 63
            do_write = act & (~stay)
            oidx = jnp.where(do_write, pos, _T)
            plsc.store_scatter(outv, [jnp.full((16,), oidx, jnp.int32)],
                               jnp.full((16,), clsb, jnp.int32), mask=lane0)
            pos = pos - do_write.astype(jnp.int32)
            w = jnp.where(act, srcb, w)
            return (w, pos)

        lax.fori_loop(0, _T, body, (bw, blen - 1))
        pltpu.sync_copy(outv.at[pl.ds(0, _T)], out_hbm.at[b])

    return bt(bp2, best)


def _backtrace_plain(bp, best):
    bw = best[:, 0]
    blen = best[:, 1]
    length = best[:, 2]
    bseq = jnp.transpose(bp, (1, 0, 2))

    def body(i, carry):
        w, pos, out = carry
        t = _T - 1 - i
        e = bseq[jnp.arange(_B), t, w]
        act = t < length
        stay = (e & 1) == 1
        srcb = e >> 8
        clsb = (e >> 1) & 63
        do_write = act & (~stay)
        out = jnp.where(
            do_write[:, None] & (jnp.arange(_T)[None, :] == pos[:, None]),
            clsb[:, None], out)
        pos = pos - do_write.astype(jnp.int32)
        w = jnp.where(act, srcb, w)
        return (w, pos, out)

    out0 = jnp.zeros((_B, _T), jnp.int32)
    _, _, out = jax.lax.fori_loop(0, _T, body, (bw, blen - 1, out0))
    return out


def kernel(data, data_length):
    bp, best = _forward(data, data_length)
    bp2 = jnp.transpose(bp, (1, 0, 2)).reshape(_B, _T * _W)
    return _sc_backtrace(bp2, best)


# speedup vs baseline: 5.5235x; 2.9631x over previous
import functools

import jax
import jax.numpy as jnp
from jax import lax
from jax.experimental import pallas as pl
from jax.experimental.pallas import tpu as pltpu
from jax.experimental.pallas import tpu_sc as plsc

_BLANK = 0
_W = 8
_NEG_INF = -1.0e30
_PAD = -2.0e30
_MASKED = -3.0e30
_BIGI = 1 << 20
_T = 512
_B = 32
_C = 64
_K = 13
_TB = 8


def _pre_body(data_ref, kv_ref, ki_ref):
    iota0 = jax.lax.broadcasted_iota(jnp.int32, (_C, _B), 0)
    for tt in range(_TB):
        x = data_ref[tt]
        x = jnp.where(iota0 == _BLANK, _MASKED, x)
        kv_rows, ki_rows = [], []
        for _ in range(_K):
            m = jnp.max(x, axis=0, keepdims=True)
            ci = jnp.min(jnp.where(x == m, iota0, _BIGI), axis=0, keepdims=True)
            kv_rows.append(m)
            ki_rows.append(ci)
            x = jnp.where(iota0 == ci, _MASKED, x)
        for _ in range(16 - _K):
            kv_rows.append(jnp.full((1, _B), _PAD, jnp.float32))
            ki_rows.append(jnp.zeros((1, _B), jnp.int32))
        kv_ref[tt] = jnp.concatenate(kv_rows, axis=0)
        ki_ref[tt] = jnp.concatenate(ki_rows, axis=0)


def _prepass(data_t):
    return pl.pallas_call(
        _pre_body,
        grid=(_T // _TB,),
        in_specs=[pl.BlockSpec((_TB, _C, _B), lambda t: (t, 0, 0))],
        out_specs=[
            pl.BlockSpec((_TB, 16, _B), lambda t: (t, 0, 0)),
            pl.BlockSpec((_TB, 16, _B), lambda t: (t, 0, 0)),
        ],
        out_shape=[
            jax.ShapeDtypeStruct((_T, 16, _B), jnp.float32),
            jax.ShapeDtypeStruct((_T, 16, _B), jnp.int32),
        ],
    )(data_t)


def _fwd_body(data_ref, dl_ref, kv_ref, ki_ref, bp_ref, best_ref,
              lpb_s, lpnb_s, lens_s, last_s):
    t = pl.program_id(0)

    @pl.when(t == 0)
    def _init():
        row = jax.lax.broadcasted_iota(jnp.int32, (_W, _B), 0)
        lpb_s[...] = jnp.where(row == 0, 0.0, _NEG_INF).astype(jnp.float32)
        lpnb_s[...] = jnp.full((_W, _B), _NEG_INF, jnp.float32)
        lens_s[...] = jnp.zeros((_W, _B), jnp.int32)
        last_s[...] = jnp.zeros((_W, _B), jnp.int32)

    lp = data_ref[0]
    kv = kv_ref[0]
    ki = ki_ref[0]
    lpb = lpb_s[...]
    lpnb = lpnb_s[...]
    lens = lens_s[...]
    last = last_s[...]

    iota064 = jax.lax.broadcasted_iota(jnp.int32, (_C, _B), 0)
    w_iota = jax.lax.broadcasted_iota(jnp.int32, (_W, _B), 0)
    hp = lens > 0
    tot = jnp.logaddexp(lpb, lpnb)
    stay_lpb = tot + lp[0:1, :]

    lp_last_rows = []
    for w in range(_W):
        match_w = iota064 == last[w : w + 1, :]
        lp_last_rows.append(
            jnp.sum(jnp.where(match_w, lp, 0.0), axis=0, keepdims=True))
    lp_last = jnp.concatenate(lp_last_rows, axis=0)

    stay_lpnb = jnp.where(hp, lpnb + lp_last, _NEG_INF)
    stay_score = jnp.logaddexp(stay_lpb, stay_lpnb)

    pieces_v = [stay_score]
    pieces_f = [w_iota]
    for w in range(_W):
        dup = (ki == last[w : w + 1, :]) & hp[w : w + 1, :]
        pieces_v.append(jnp.where(dup, _PAD, tot[w : w + 1, :] + kv))
        pieces_f.append(8 + w * _C + ki)
    pieces_v.append(jnp.where(hp, lpb + lp_last, _PAD))
    pieces_f.append(jnp.where(hp, 8 + w_iota * _C + last, 2 * _BIGI + w_iota))
    cand = jnp.concatenate(pieces_v, axis=0)
    fidx = jnp.concatenate(pieces_f, axis=0)

    sel_v, sel_f = [], []
    v = cand
    for _ in range(_W):
        m = jnp.max(v, axis=0, keepdims=True)
        fk = jnp.min(jnp.where(v == m, fidx, _BIGI), axis=0, keepdims=True)
        sel_v.append(m)
        sel_f.append(fk)
        v = jnp.where(fidx == fk, _MASKED, v)
    top_v = jnp.concatenate(sel_v, axis=0)
    top_f = jnp.concatenate(sel_f, axis=0)

    is_stay = top_f < _W
    src = jnp.where(is_stay, top_f, (top_f - _W) >> 6)
    cls = jnp.where(is_stay, 0, (top_f - _W) & 63)

    def sel8(s, arr):
        acc = jnp.broadcast_to(arr[0:1, :], s.shape)
        for w in range(1, _W):
            acc = jnp.where(s == w, arr[w : w + 1, :], acc)
        return acc

    g_staylpb = sel8(src, stay_lpb)
    g_staylpnb = sel8(src, stay_lpnb)
    g_lens = sel8(src, lens)
    g_last = sel8(src, last)

    new_lpb = jnp.where(is_stay, g_staylpb, _NEG_INF)
    new_lpnb = jnp.where(is_stay, g_staylpnb, top_v)
    new_lens = jnp.where(is_stay, g_lens, jnp.minimum(g_lens + 1, _T))
    new_last = jnp.where(is_stay, g_last, cls)

    active = t < dl_ref[...]
    lpb_s[...] = jnp.where(active, new_lpb, lpb)
    lpnb_s[...] = jnp.where(active, new_lpnb, lpnb)
    lens_s[...] = jnp.where(active, new_lens, lens)
    last_s[...] = jnp.where(active, new_last, last)

    bp_ref[...] = ((src << 8) | (cls << 1) | is_stay.astype(jnp.int32)).reshape(1, _W, _B)

    @pl.when(t == _T - 1)
    def _final():
        fscore = jnp.logaddexp(lpb_s[...], lpnb_s[...])
        m = jnp.max(fscore, axis=0, keepdims=True)
        bw = jnp.min(jnp.where(fscore == m, w_iota, _BIGI), axis=0, keepdims=True)
        blen = sel8(bw, lens_s[...])
        zpad = jnp.zeros((13, _B), jnp.int32)
        best_ref[...] = jnp.concatenate(
            [bw, blen, dl_ref[...][0:1, :], zpad], axis=0)


def _forward(data, data_length):
    data_t = jnp.transpose(data, (0, 2, 1))
    dlt = jnp.broadcast_to(data_length[None, :], (_W, _B))
    kv, ki = _prepass(data_t)
    bp, best = pl.pallas_call(
        _fwd_body,
        grid=(_T,),
        in_specs=[
            pl.BlockSpec((1, _C, _B), lambda t: (t, 0, 0)),
            pl.BlockSpec((_W, _B), lambda t: (0, 0)),
            pl.BlockSpec((1, 16, _B), lambda t: (t, 0, 0)),
            pl.BlockSpec((1, 16, _B), lambda t: (t, 0, 0)),
        ],
        out_specs=[
            pl.BlockSpec((1, _W, _B), lambda t: (t, 0, 0)),
            pl.BlockSpec((16, _B), lambda t: (0, 0)),
        ],
        out_shape=[
            jax.ShapeDtypeStruct((_T, _W, _B), jnp.int32),
            jax.ShapeDtypeStruct((16, _B), jnp.int32),
        ],
        scratch_shapes=[
            pltpu.VMEM((_W, _B), jnp.float32),
            pltpu.VMEM((_W, _B), jnp.float32),
            pltpu.VMEM((_W, _B), jnp.int32),
            pltpu.VMEM((_W, _B), jnp.int32),
        ],
    )(data_t, dlt, kv, ki)
    return bp, best


def _sc_backtrace(bp2, best):
    mesh = plsc.VectorSubcoreMesh(core_axis_name="c", subcore_axis_name="s")

    @functools.partial(
        pl.kernel,
        mesh=mesh,
        out_type=jax.ShapeDtypeStruct((_B, _T), jnp.int32),
        compiler_params=pltpu.CompilerParams(needs_layout_passes=False),
        scratch_types=[
            pltpu.VMEM((_T * _W,), jnp.int32),
            pltpu.VMEM((528,), jnp.int32),
            pltpu.VMEM((16,), jnp.int32),
        ],
    )
    def bt(bp_hbm, info_hbm, out_hbm, bpv, outv, infov):
        b = lax.axis_index("s") * 2 + lax.axis_index("c")
        pltpu.sync_copy(bp_hbm.at[b], bpv)
        pltpu.sync_copy(info_hbm.at[b], infov)
        z = jnp.zeros((16,), jnp.int32)
        for i in range(33):
            outv[pl.ds(i * 16, 16)] = z
        iv = infov[...]
        bw = iv[0]
        blen = iv[1]
        ln = iv[2]
        lane0 = lax.iota(jnp.int32, 16) == 0

        def body(i, carry):
            w, pos = carry
            t = _T - 1 - i
            idxv = jnp.full((16,), t * _W + w, jnp.int32)
            e = plsc.load_gather(bpv, [idxv])[0]
            act = t < ln
            stay = (e & 1) == 1
            srcb = e >> 8
            clsb = (e >> 1) & 63
            do_write = act & (~stay)
            oidx = jnp.where(do_write, pos, _T)
            plsc.store_scatter(outv, [jnp.full((16,), oidx, jnp.int32)],
                               jnp.full((16,), clsb, jnp.int32), mask=lane0)
            pos = pos - do_write.astype(jnp.int32)
            w = jnp.where(act, srcb, w)
            return (w, pos)

        lax.fori_loop(0, _T, body, (bw, blen - 1))
        pltpu.sync_copy(outv.at[pl.ds(0, _T)], out_hbm.at[b])

    return bt(bp2, best)


def _backtrace_plain(bp, best):
    bw = best[0]
    blen = best[1]
    length = best[2]
    bseq = jnp.transpose(bp, (2, 0, 1))

    def body(i, carry):
        w, pos, out = carry
        t = _T - 1 - i
        e = bseq[jnp.arange(_B), t, w]
        act = t < length
        stay = (e & 1) == 1
        srcb = e >> 8
        clsb = (e >> 1) & 63
        do_write = act & (~stay)
        out = jnp.where(
            do_write[:, None] & (jnp.arange(_T)[None, :] == pos[:, None]),
            clsb[:, None], out)
        pos = pos - do_write.astype(jnp.int32)
        w = jnp.where(act, srcb, w)
        return (w, pos, out)

    out0 = jnp.zeros((_B, _T), jnp.int32)
    _, _, out = jax.lax.fori_loop(0, _T, body, (bw, blen - 1, out0))
    return out


def kernel(data, data_length):
    bp, best = _forward(data, data_length)
    bp2 = jnp.transpose(bp, (2, 0, 1)).reshape(_B, _T * _W)
    info = jnp.transpose(best)
    return _sc_backtrace(bp2, info)


# speedup vs baseline: 7.0034x; 1.2679x over previous
import functools

import jax
import jax.numpy as jnp
from jax import lax
from jax.experimental import pallas as pl
from jax.experimental.pallas import tpu as pltpu
from jax.experimental.pallas import tpu_sc as plsc

_BLANK = 0
_W = 8
_NEG_INF = -1.0e30
_PAD = -2.0e30
_MASKED = -3.0e30
_BIGI = 1 << 20
_T = 512
_B = 32
_C = 64
_K = 13
_TB = 8


def _pre_body(data_ref, kv_ref, ki_ref):
    iota0 = jax.lax.broadcasted_iota(jnp.int32, (_C, _B), 0)
    for tt in range(_TB):
        x = data_ref[tt]
        x = jnp.where(iota0 == _BLANK, _MASKED, x)
        kv_rows, ki_rows = [], []
        for _ in range(_K):
            m = jnp.max(x, axis=0, keepdims=True)
            ci = jnp.min(jnp.where(x == m, iota0, _BIGI), axis=0, keepdims=True)
            kv_rows.append(m)
            ki_rows.append(ci)
            x = jnp.where(iota0 == ci, _MASKED, x)
        for _ in range(16 - _K):
            kv_rows.append(jnp.full((1, _B), _PAD, jnp.float32))
            ki_rows.append(jnp.zeros((1, _B), jnp.int32))
        kv_ref[tt] = jnp.concatenate(kv_rows, axis=0)
        ki_ref[tt] = jnp.concatenate(ki_rows, axis=0)


def _prepass(data_t):
    return pl.pallas_call(
        _pre_body,
        grid=(_T // _TB,),
        in_specs=[pl.BlockSpec((_TB, _C, _B), lambda t: (t, 0, 0))],
        out_specs=[
            pl.BlockSpec((_TB, 16, _B), lambda t: (t, 0, 0)),
            pl.BlockSpec((_TB, 16, _B), lambda t: (t, 0, 0)),
        ],
        out_shape=[
            jax.ShapeDtypeStruct((_T, 16, _B), jnp.float32),
            jax.ShapeDtypeStruct((_T, 16, _B), jnp.int32),
        ],
    )(data_t)


def sel8(s, arr):
    acc = jnp.broadcast_to(arr[0:1, :], s.shape)
    for w in range(1, _W):
        acc = jnp.where(s == w, arr[w : w + 1, :], acc)
    return acc


def _step(t, lp, kv, ki, dl, state):
    lpb, lpnb, lens, last = state
    iota064 = jax.lax.broadcasted_iota(jnp.int32, (_C, _B), 0)
    w_iota = jax.lax.broadcasted_iota(jnp.int32, (_W, _B), 0)
    hp = lens > 0
    tot = jnp.logaddexp(lpb, lpnb)
    stay_lpb = tot + lp[0:1, :]

    lp_last_rows = []
    for w in range(_W):
        match_w = iota064 == last[w : w + 1, :]
        lp_last_rows.append(
            jnp.sum(jnp.where(match_w, lp, 0.0), axis=0, keepdims=True))
    lp_last = jnp.concatenate(lp_last_rows, axis=0)

    stay_lpnb = jnp.where(hp, lpnb + lp_last, _NEG_INF)
    stay_score = jnp.logaddexp(stay_lpb, stay_lpnb)

    pieces_v = [stay_score]
    pieces_f = [w_iota]
    for w in range(_W):
        dup = (ki == last[w : w + 1, :]) & hp[w : w + 1, :]
        pieces_v.append(jnp.where(dup, _PAD, tot[w : w + 1, :] + kv))
        pieces_f.append(8 + w * _C + ki)
    pieces_v.append(jnp.where(hp, lpb + lp_last, _PAD))
    pieces_f.append(jnp.where(hp, 8 + w_iota * _C + last, 2 * _BIGI + w_iota))
    cand = jnp.concatenate(pieces_v, axis=0)
    fidx = jnp.concatenate(pieces_f, axis=0)

    sel_v, sel_f = [], []
    v = cand
    for _ in range(_W):
        m = jnp.max(v, axis=0, keepdims=True)
        fk = jnp.min(jnp.where(v == m, fidx, _BIGI), axis=0, keepdims=True)
        sel_v.append(m)
        sel_f.append(fk)
        v = jnp.where(fidx == fk, _MASKED, v)
    top_v = jnp.concatenate(sel_v, axis=0)
    top_f = jnp.concatenate(sel_f, axis=0)

    is_stay = top_f < _W
    src = jnp.where(is_stay, top_f, (top_f - _W) >> 6)
    cls = jnp.where(is_stay, 0, (top_f - _W) & 63)

    g_staylpb = sel8(src, stay_lpb)
    g_staylpnb = sel8(src, stay_lpnb)
    g_lens = sel8(src, lens)
    g_last = sel8(src, last)

    new_lpb = jnp.where(is_stay, g_staylpb, _NEG_INF)
    new_lpnb = jnp.where(is_stay, g_staylpnb, top_v)
    new_lens = jnp.where(is_stay, g_lens, jnp.minimum(g_lens + 1, _T))
    new_last = jnp.where(is_stay, g_last, cls)

    active = t < dl
    state = (
        jnp.where(active, new_lpb, lpb),
        jnp.where(active, new_lpnb, lpnb),
        jnp.where(active, new_lens, lens),
        jnp.where(active, new_last, last),
    )
    bpw = (src << 8) | (cls << 1) | is_stay.astype(jnp.int32)
    return state, bpw


_TS = 4


def _fwd_body(data_ref, dl_ref, kv_ref, ki_ref, bp_ref, best_ref,
              lpb_s, lpnb_s, lens_s, last_s):
    pid = pl.program_id(0)

    @pl.when(pid == 0)
    def _init():
        row = jax.lax.broadcasted_iota(jnp.int32, (_W, _B), 0)
        lpb_s[...] = jnp.where(row == 0, 0.0, _NEG_INF).astype(jnp.float32)
        lpnb_s[...] = jnp.full((_W, _B), _NEG_INF, jnp.float32)
        lens_s[...] = jnp.zeros((_W, _B), jnp.int32)
        last_s[...] = jnp.zeros((_W, _B), jnp.int32)

    dl = dl_ref[...]
    state = (lpb_s[...], lpnb_s[...], lens_s[...], last_s[...])
    bpws = []
    for tt in range(_TS):
        t = pid * _TS + tt
        state, bpw = _step(t, data_ref[tt], kv_ref[tt], ki_ref[tt], dl, state)
        bpws.append(bpw.reshape(1, _W, _B))
    lpb_s[...], lpnb_s[...], lens_s[...], last_s[...] = state
    bp_ref[...] = jnp.concatenate(bpws, axis=0)

    @pl.when(pid == _T // _TS - 1)
    def _final():
        w_iota = jax.lax.broadcasted_iota(jnp.int32, (_W, _B), 0)
        fscore = jnp.logaddexp(state[0], state[1])
        m = jnp.max(fscore, axis=0, keepdims=True)
        bw = jnp.min(jnp.where(fscore == m, w_iota, _BIGI), axis=0, keepdims=True)
        blen = sel8(bw, state[2])
        zpad = jnp.zeros((13, _B), jnp.int32)
        best_ref[...] = jnp.concatenate([bw, blen, dl[0:1, :], zpad], axis=0)


def _forward(data, data_length):
    data_t = jnp.transpose(data, (0, 2, 1))
    dlt = jnp.broadcast_to(data_length[None, :], (_W, _B))
    kv, ki = _prepass(data_t)
    bp, best = pl.pallas_call(
        _fwd_body,
        grid=(_T // _TS,),
        in_specs=[
            pl.BlockSpec((_TS, _C, _B), lambda t: (t, 0, 0)),
            pl.BlockSpec((_W, _B), lambda t: (0, 0)),
            pl.BlockSpec((_TS, 16, _B), lambda t: (t, 0, 0)),
            pl.BlockSpec((_TS, 16, _B), lambda t: (t, 0, 0)),
        ],
        out_specs=[
            pl.BlockSpec((_TS, _W, _B), lambda t: (t, 0, 0)),
            pl.BlockSpec((16, _B), lambda t: (0, 0)),
        ],
        out_shape=[
            jax.ShapeDtypeStruct((_T, _W, _B), jnp.int32),
            jax.ShapeDtypeStruct((16, _B), jnp.int32),
        ],
        scratch_shapes=[
            pltpu.VMEM((_W, _B), jnp.float32),
            pltpu.VMEM((_W, _B), jnp.float32),
            pltpu.VMEM((_W, _B), jnp.int32),
            pltpu.VMEM((_W, _B), jnp.int32),
        ],
    )(data_t, dlt, kv, ki)
    return bp, best


def _sc_backtrace(bp2, best):
    mesh = plsc.VectorSubcoreMesh(core_axis_name="c", subcore_axis_name="s")

    @functools.partial(
        pl.kernel,
        mesh=mesh,
        out_type=jax.ShapeDtypeStruct((_B, _T), jnp.int32),
        compiler_params=pltpu.CompilerParams(needs_layout_passes=False),
        scratch_types=[
            pltpu.VMEM((_T * _W,), jnp.int32),
            pltpu.VMEM((528,), jnp.int32),
            pltpu.VMEM((16,), jnp.int32),
        ],
    )
    def bt(bp_hbm, info_hbm, out_hbm, bpv, outv, infov):
        b = lax.axis_index("s") * 2 + lax.axis_index("c")
        pltpu.sync_copy(bp_hbm.at[b], bpv)
        pltpu.sync_copy(info_hbm.at[b], infov)
        z = jnp.zeros((16,), jnp.int32)
        for i in range(33):
            outv[pl.ds(i * 16, 16)] = z
        iv = infov[...]
        bw = iv[0]
        blen = iv[1]
        ln = iv[2]
        lane0 = lax.iota(jnp.int32, 16) == 0

        def body(i, carry):
            w, pos = carry
            t = _T - 1 - i
            idxv = jnp.full((16,), t * _W + w, jnp.int32)
            e = plsc.load_gather(bpv, [idxv])[0]
            act = t < ln
            stay = (e & 1) == 1
            srcb = e >> 8
            clsb = (e >> 1) & 63
            do_write = act & (~stay)
            oidx = jnp.where(do_write, pos, _T)
            plsc.store_scatter(outv, [jnp.full((16,), oidx, jnp.int32)],
                               jnp.full((16,), clsb, jnp.int32), mask=lane0)
            pos = pos - do_write.astype(jnp.int32)
            w = jnp.where(act, srcb, w)
            return (w, pos)

        lax.fori_loop(0, _T, body, (bw, blen - 1))
        pltpu.sync_copy(outv.at[pl.ds(0, _T)], out_hbm.at[b])

    return bt(bp2, best)


def _backtrace_plain(bp, best):
    bw = best[0]
    blen = best[1]
    length = best[2]
    bseq = jnp.transpose(bp, (2, 0, 1))

    def body(i, carry):
        w, pos, out = carry
        t = _T - 1 - i
        e = bseq[jnp.arange(_B), t, w]
        act = t < length
        stay = (e & 1) == 1
        srcb = e >> 8
        clsb = (e >> 1) & 63
        do_write = act & (~stay)
        out = jnp.where(
            do_write[:, None] & (jnp.arange(_T)[None, :] == pos[:, None]),
            clsb[:, None], out)
        pos = pos - do_write.astype(jnp.int32)
        w = jnp.where(act, srcb, w)
        return (w, pos, out)

    out0 = jnp.zeros((_B, _T), jnp.int32)
    _, _, out = jax.lax.fori_loop(0, _T, body, (bw, blen - 1, out0))
    return out


def kernel(data, data_length):
    bp, best = _forward(data, data_length)
    bp2 = jnp.transpose(bp, (2, 0, 1)).reshape(_B, _T * _W)
    info = jnp.transpose(best)
    return _sc_backtrace(bp2, info)


# speedup vs baseline: 7.1295x; 1.0180x over previous
import functools

import jax
import jax.numpy as jnp
from jax import lax
from jax.experimental import pallas as pl
from jax.experimental.pallas import tpu as pltpu
from jax.experimental.pallas import tpu_sc as plsc

_BLANK = 0
_W = 8
_NEG_INF = -1.0e30
_PAD = -2.0e30
_MASKED = -3.0e30
_BIGI = 1 << 20
_T = 512
_B = 32
_C = 64
_K = 13
_TB = 16


def _pre_body(data_ref, kv_ref, ki_ref):
    iota0 = jax.lax.broadcasted_iota(jnp.int32, (_C, _B), 0)
    for tt in range(_TB):
        x = data_ref[tt]
        x = jnp.where(iota0 == _BLANK, _MASKED, x)
        kv_rows, ki_rows = [], []
        for _ in range(_K):
            m = jnp.max(x, axis=0, keepdims=True)
            ci = jnp.min(jnp.where(x == m, iota0, _BIGI), axis=0, keepdims=True)
            kv_rows.append(m)
            ki_rows.append(ci)
            x = jnp.where(iota0 == ci, _MASKED, x)
        for _ in range(16 - _K):
            kv_rows.append(jnp.full((1, _B), _PAD, jnp.float32))
            ki_rows.append(jnp.zeros((1, _B), jnp.int32))
        kv_ref[tt] = jnp.concatenate(kv_rows, axis=0)
        ki_ref[tt] = jnp.concatenate(ki_rows, axis=0)


def _prepass(data_t):
    return pl.pallas_call(
        _pre_body,
        grid=(_T // _TB,),
        in_specs=[pl.BlockSpec((_TB, _C, _B), lambda t: (t, 0, 0))],
        out_specs=[
            pl.BlockSpec((_TB, 16, _B), lambda t: (t, 0, 0)),
            pl.BlockSpec((_TB, 16, _B), lambda t: (t, 0, 0)),
        ],
        out_shape=[
            jax.ShapeDtypeStruct((_T, 16, _B), jnp.float32),
            jax.ShapeDtypeStruct((_T, 16, _B), jnp.int32),
        ],
    )(data_t)


def sel8(s, arr):
    acc = jnp.broadcast_to(arr[0:1, :], s.shape)
    for w in range(1, _W):
        acc = jnp.where(s == w, arr[w : w + 1, :], acc)
    return acc


def _step(t, lp, kv, ki, dl, state):
    lpb, lpnb, lens, last = state
    iota064 = jax.lax.broadcasted_iota(jnp.int32, (_C, _B), 0)
    w_iota = jax.lax.broadcasted_iota(jnp.int32, (_W, _B), 0)
    hp = lens > 0
    tot = jnp.logaddexp(lpb, lpnb)
    stay_lpb = tot + lp[0:1, :]

    lp_last_rows = []
    for w in range(_W):
        match_w = iota064 == last[w : w + 1, :]
        lp_last_rows.append(
            jnp.sum(jnp.where(match_w, lp, 0.0), axis=0, keepdims=True))
    lp_last = jnp.concatenate(lp_last_rows, axis=0)

    stay_lpnb = jnp.where(hp, lpnb + lp_last, _NEG_INF)
    stay_score = jnp.logaddexp(stay_lpb, stay_lpnb)

    pieces_v = [stay_score]
    pieces_f = [w_iota]
    for w in range(_W):
        dup = (ki == last[w : w + 1, :]) & hp[w : w + 1, :]
        pieces_v.append(jnp.where(dup, _PAD, tot[w : w + 1, :] + kv))
        pieces_f.append(8 + w * _C + ki)
    pieces_v.append(jnp.where(hp, lpb + lp_last, _PAD))
    pieces_f.append(jnp.where(hp, 8 + w_iota * _C + last, 2 * _BIGI + w_iota))
    cand = jnp.concatenate(pieces_v, axis=0)
    fidx = jnp.concatenate(pieces_f, axis=0)

    sel_v, sel_f = [], []
    v = cand
    for _ in range(_W):
        m = jnp.max(v, axis=0, keepdims=True)
        fk = jnp.min(jnp.where(v == m, fidx, _BIGI), axis=0, keepdims=True)
        sel_v.append(m)
        sel_f.append(fk)
        v = jnp.where(fidx == fk, _MASKED, v)
    top_v = jnp.concatenate(sel_v, axis=0)
    top_f = jnp.concatenate(sel_f, axis=0)

    is_stay = top_f < _W
    src = jnp.where(is_stay, top_f, (top_f - _W) >> 6)
    cls = jnp.where(is_stay, 0, (top_f - _W) & 63)

    g_staylpb = sel8(src, stay_lpb)
    g_staylpnb = sel8(src, stay_lpnb)
    g_lens = sel8(src, lens)
    g_last = sel8(src, last)

    new_lpb = jnp.where(is_stay, g_staylpb, _NEG_INF)
    new_lpnb = jnp.where(is_stay, g_staylpnb, top_v)
    new_lens = jnp.where(is_stay, g_lens, jnp.minimum(g_lens + 1, _T))
    new_last = jnp.where(is_stay, g_last, cls)

    active = t < dl
    state = (
        jnp.where(active, new_lpb, lpb),
        jnp.where(active, new_lpnb, lpnb),
        jnp.where(active, new_lens, lens),
        jnp.where(active, new_last, last),
    )
    bpw = (src << 8) | (cls << 1) | is_stay.astype(jnp.int32)
    return state, bpw


_TS = 8


def _fwd_body(data_ref, dl_ref, kv_ref, ki_ref, bp_ref, best_ref,
              lpb_s, lpnb_s, lens_s, last_s):
    pid = pl.program_id(0)

    @pl.when(pid == 0)
    def _init():
        row = jax.lax.broadcasted_iota(jnp.int32, (_W, _B), 0)
        lpb_s[...] = jnp.where(row == 0, 0.0, _NEG_INF).astype(jnp.float32)
        lpnb_s[...] = jnp.full((_W, _B), _NEG_INF, jnp.float32)
        lens_s[...] = jnp.zeros((_W, _B), jnp.int32)
        last_s[...] = jnp.zeros((_W, _B), jnp.int32)

    dl = dl_ref[...]
    state = (lpb_s[...], lpnb_s[...], lens_s[...], last_s[...])
    bpws = []
    for tt in range(_TS):
        t = pid * _TS + tt
        state, bpw = _step(t, data_ref[tt], kv_ref[tt], ki_ref[tt], dl, state)
        bpws.append(bpw.reshape(1, _W, _B))
    lpb_s[...], lpnb_s[...], lens_s[...], last_s[...] = state
    bp_ref[...] = jnp.concatenate(bpws, axis=0)

    @pl.when(pid == _T // _TS - 1)
    def _final():
        w_iota = jax.lax.broadcasted_iota(jnp.int32, (_W, _B), 0)
        fscore = jnp.logaddexp(state[0], state[1])
        m = jnp.max(fscore, axis=0, keepdims=True)
        bw = jnp.min(jnp.where(fscore == m, w_iota, _BIGI), axis=0, keepdims=True)
        blen = sel8(bw, state[2])
        zpad = jnp.zeros((13, _B), jnp.int32)
        best_ref[...] = jnp.concatenate([bw, blen, dl[0:1, :], zpad], axis=0)


def _forward(data, data_length):
    data_t = jnp.transpose(data, (0, 2, 1))
    dlt = jnp.broadcast_to(data_length[None, :], (_W, _B))
    kv, ki = _prepass(data_t)
    bp, best = pl.pallas_call(
        _fwd_body,
        grid=(_T // _TS,),
        in_specs=[
            pl.BlockSpec((_TS, _C, _B), lambda t: (t, 0, 0)),
            pl.BlockSpec((_W, _B), lambda t: (0, 0)),
            pl.BlockSpec((_TS, 16, _B), lambda t: (t, 0, 0)),
            pl.BlockSpec((_TS, 16, _B), lambda t: (t, 0, 0)),
        ],
        out_specs=[
            pl.BlockSpec((_TS, _W, _B), lambda t: (t, 0, 0)),
            pl.BlockSpec((16, _B), lambda t: (0, 0)),
        ],
        out_shape=[
            jax.ShapeDtypeStruct((_T, _W, _B), jnp.int32),
            jax.ShapeDtypeStruct((16, _B), jnp.int32),
        ],
        scratch_shapes=[
            pltpu.VMEM((_W, _B), jnp.float32),
            pltpu.VMEM((_W, _B), jnp.float32),
            pltpu.VMEM((_W, _B), jnp.int32),
            pltpu.VMEM((_W, _B), jnp.int32),
        ],
    )(data_t, dlt, kv, ki)
    return bp, best


def _sc_backtrace(bp2, best):
    mesh = plsc.VectorSubcoreMesh(core_axis_name="c", subcore_axis_name="s")

    @functools.partial(
        pl.kernel,
        mesh=mesh,
        out_type=jax.ShapeDtypeStruct((_B, _T), jnp.int32),
        compiler_params=pltpu.CompilerParams(needs_layout_passes=False),
        scratch_types=[
            pltpu.VMEM((_T * _W,), jnp.int32),
            pltpu.VMEM((528,), jnp.int32),
            pltpu.VMEM((16,), jnp.int32),
        ],
    )
    def bt(bp_hbm, info_hbm, out_hbm, bpv, outv, infov):
        b = lax.axis_index("s") * 2 + lax.axis_index("c")
        pltpu.sync_copy(bp_hbm.at[b], bpv)
        pltpu.sync_copy(info_hbm.at[b], infov)
        z = jnp.zeros((16,), jnp.int32)
        for i in range(33):
            outv[pl.ds(i * 16, 16)] = z
        iv = infov[...]
        bw = iv[0]
        blen = iv[1]
        ln = iv[2]
        lane0 = lax.iota(jnp.int32, 16) == 0

        def body(i, carry):
            w, pos = carry
            t = _T - 1 - i
            idxv = jnp.full((16,), t * _W + w, jnp.int32)
            e = plsc.load_gather(bpv, [idxv])[0]
            act = t < ln
            stay = (e & 1) == 1
            srcb = e >> 8
            clsb = (e >> 1) & 63
            do_write = act & (~stay)
            oidx = jnp.where(do_write, pos, _T)
            plsc.store_scatter(outv, [jnp.full((16,), oidx, jnp.int32)],
                               jnp.full((16,), clsb, jnp.int32), mask=lane0)
            pos = pos - do_write.astype(jnp.int32)
            w = jnp.where(act, srcb, w)
            return (w, pos)

        lax.fori_loop(0, _T, body, (bw, blen - 1))
        pltpu.sync_copy(outv.at[pl.ds(0, _T)], out_hbm.at[b])

    return bt(bp2, best)


def _backtrace_plain(bp, best):
    bw = best[0]
    blen = best[1]
    length = best[2]
    bseq = jnp.transpose(bp, (2, 0, 1))

    def body(i, carry):
        w, pos, out = carry
        t = _T - 1 - i
        e = bseq[jnp.arange(_B), t, w]
        act = t < length
        stay = (e & 1) == 1
        srcb = e >> 8
        clsb = (e >> 1) & 63
        do_write = act & (~stay)
        out = jnp.where(
            do_write[:, None] & (jnp.arange(_T)[None, :] == pos[:, None]),
            clsb[:, None], out)
        pos = pos - do_write.astype(jnp.int32)
        w = jnp.where(act, srcb, w)
        return (w, pos, out)

    out0 = jnp.zeros((_B, _T), jnp.int32)
    _, _, out = jax.lax.fori_loop(0, _T, body, (bw, blen - 1, out0))
    return out


def kernel(data, data_length):
    bp, best = _forward(data, data_length)
    bp2 = jnp.transpose(bp, (2, 0, 1)).reshape(_B, _T * _W)
    info = jnp.transpose(best)
    return _sc_backtrace(bp2, info)
